# baseline (speedup 1.0000x reference)
import functools

import jax
import jax.numpy as jnp
from jax import lax
from jax.experimental import pallas as pl
from jax.experimental.pallas import tpu as pltpu
from jax.experimental.pallas import tpu_sc as plsc


def _make_sc_kernel(num_out, num_in, num_cb, igs, cb_size):
    info = plsc.get_sparse_core_info()
    nc, ns, L = info.num_cores, info.num_subcores, info.num_lanes
    nw = nc * ns
    rows_per_w = num_out // nw
    flat_cb_len = num_cb * cb_size * igs
    num_in_elems = num_in * igs
    codes_row_len = num_in * num_cb
    blocks = num_in // L

    mesh = plsc.VectorSubcoreMesh(core_axis_name="c", subcore_axis_name="s")

    @functools.partial(
        pl.kernel,
        mesh=mesh,
        out_type=jax.ShapeDtypeStruct((num_out, num_in_elems), jnp.float32),
        scratch_types=[
            pltpu.VMEM((flat_cb_len // 2,), jnp.int32),
            pltpu.VMEM((rows_per_w,), jnp.float32),
            pltpu.VMEM((2, codes_row_len), jnp.int32),
            pltpu.VMEM((2, num_in_elems), jnp.float32),
            pltpu.SemaphoreType.DMA,
            pltpu.SemaphoreType.DMA,
            pltpu.SemaphoreType.DMA,
            pltpu.SemaphoreType.DMA,
        ],
        compiler_params=pltpu.CompilerParams(needs_layout_passes=False),
    )
    def k(codes_hbm, cb_hbm, scales_hbm, out_hbm, cb_v, sc_v, codes_v, out_v,
          si0, si1, so0, so1):
        wid = lax.axis_index("s") * nc + lax.axis_index("c")
        row0 = wid * rows_per_w
        pltpu.sync_copy(cb_hbm, cb_v)
        pltpu.sync_copy(scales_hbm.at[pl.ds(row0, rows_per_w)], sc_v)

        lane = lax.iota(jnp.int32, L)
        scat_pats = [lane * igs + j for j in range(igs)]
        sis = (si0, si1)
        sos = (so0, so1)

        def cp_in(row, b):
            return pltpu.make_async_copy(
                codes_hbm.at[row], codes_v.at[b], sis[b])

        def cp_out(row, b):
            return pltpu.make_async_copy(
                out_v.at[b], out_hbm.at[row], sos[b])

        cp_in(row0, 0).start()
        cp_in(row0 + 1, 1).start()
        pairs = rows_per_w // 2

        def pair_body(p, carry):
            for b in range(2):
                r = p * 2 + b
                row = row0 + r
                cp_in(row, b).wait()

                @pl.when(p > 0)
                def _wait_out():
                    cp_out(row - 2, b).wait()

                s = plsc.load_gather(sc_v, [jnp.full((L,), r, jnp.int32)])

                @plsc.parallel_loop(0, blocks, unroll=4)
                def blk_body(ib):
                    out_blk = out_v.at[b].at[pl.ds(ib * (L * igs), L * igs)]
                    cvecs = [
                        codes_v[b, pl.ds(m * num_in + ib * L, L)]
                        for m in range(num_cb)
                    ]
                    himask = jnp.full((L,), -65536, jnp.int32)
                    for jp in range(igs // 2):
                        acc_lo = jnp.zeros((L,), jnp.float32)
                        acc_hi = jnp.zeros((L,), jnp.float32)
                        for m in range(num_cb):
                            off = (jp * num_cb + m) * cb_size
                            w = plsc.load_gather(
                                cb_v.at[pl.ds(off, cb_size)], [cvecs[m]])
                            acc_lo = acc_lo + plsc.bitcast(w << 16, jnp.float32)
                            acc_hi = acc_hi + plsc.bitcast(w & himask, jnp.float32)
                        plsc.store_scatter(out_blk, [scat_pats[2 * jp]],
                                           acc_lo * s)
                        plsc.store_scatter(out_blk, [scat_pats[2 * jp + 1]],
                                           acc_hi * s)

                cp_out(row, b).start()

                @pl.when(p < pairs - 1)
                def _next_in():
                    cp_in(row + 2, b).start()

            return carry

        lax.fori_loop(0, pairs, pair_body, 0)
        cp_out(row0 + rows_per_w - 2, 0).wait()
        cp_out(row0 + rows_per_w - 1, 1).wait()

    return k


def kernel(codes, codebooks, scales):
    num_out, num_in, num_cb = codes.shape
    _, cb_size, ogs, igs = codebooks.shape
    codes2d = codes.transpose(0, 2, 1).reshape(num_out, num_cb * num_in)
    cbj = codebooks.reshape(num_cb, cb_size, igs)
    bits = jax.lax.bitcast_convert_type(
        cbj.astype(jnp.bfloat16), jnp.uint16).astype(jnp.uint32)
    words = bits[:, :, 0::2] | (bits[:, :, 1::2] << 16)
    flat_cb = jax.lax.bitcast_convert_type(
        words.transpose(2, 0, 1).reshape(-1), jnp.int32)
    scales1d = scales.reshape(num_out)
    k = _make_sc_kernel(num_out, num_in, num_cb, igs, cb_size)
    return k(codes2d, flat_cb, scales1d)


# speedup vs baseline: 1.3722x; 1.3722x over previous
import functools

import jax
import jax.numpy as jnp
from jax import lax
from jax.experimental import pallas as pl
from jax.experimental.pallas import tpu as pltpu
from jax.experimental.pallas import tpu_sc as plsc


def _make_sc_kernel(num_out, num_in, num_cb, igs, cb_size):
    info = plsc.get_sparse_core_info()
    nc, ns, L = info.num_cores, info.num_subcores, info.num_lanes
    nw = nc * ns
    rows_per_w = num_out // nw
    flat_cb_len = num_cb * cb_size * igs
    num_in_elems = num_in * igs
    codes_row_len = num_in * num_cb
    blocks = num_in // L

    mesh = plsc.VectorSubcoreMesh(core_axis_name="c", subcore_axis_name="s")

    @functools.partial(
        pl.kernel,
        mesh=mesh,
        out_type=jax.ShapeDtypeStruct((num_out, num_in_elems), jnp.float32),
        scratch_types=[
            pltpu.VMEM((flat_cb_len,), jnp.int32),
            pltpu.VMEM((rows_per_w,), jnp.float32),
            pltpu.VMEM((2, codes_row_len), jnp.int32),
            pltpu.VMEM((2, num_in_elems), jnp.float32),
            pltpu.SemaphoreType.DMA,
            pltpu.SemaphoreType.DMA,
            pltpu.SemaphoreType.DMA,
            pltpu.SemaphoreType.DMA,
        ],
        compiler_params=pltpu.CompilerParams(needs_layout_passes=False),
    )
    def k(codes_hbm, cb_hbm, scales_hbm, out_hbm, cb_v, sc_v, codes_v, out_v,
          si0, si1, so0, so1):
        wid = lax.axis_index("s") * nc + lax.axis_index("c")
        row0 = wid * rows_per_w
        pltpu.sync_copy(cb_hbm, cb_v)
        pltpu.sync_copy(scales_hbm.at[pl.ds(row0, rows_per_w)], sc_v)

        lane = lax.iota(jnp.int32, L)
        scat_pats = [lane * igs + j for j in range(igs)]
        sis = (si0, si1)
        sos = (so0, so1)

        def cp_in(row, b):
            return pltpu.make_async_copy(
                codes_hbm.at[row], codes_v.at[b], sis[b])

        def cp_out(row, b):
            return pltpu.make_async_copy(
                out_v.at[b], out_hbm.at[row], sos[b])

        cp_in(row0, 0).start()
        cp_in(row0 + 1, 1).start()
        pairs = rows_per_w // 2

        def pair_body(p, carry):
            for b in range(2):
                r = p * 2 + b
                row = row0 + r
                cp_in(row, b).wait()

                @pl.when(p > 0)
                def _wait_out():
                    cp_out(row - 2, b).wait()

                s = plsc.load_gather(sc_v, [jnp.full((L,), r, jnp.int32)])

                @plsc.parallel_loop(0, blocks, unroll=2)
                def blk_body(ib):
                    out_blk = out_v.at[b].at[pl.ds(ib * (L * igs), L * igs)]
                    hl = lane >> 3
                    cvecs = [
                        (codes_v[b, pl.ds(m * num_in + ib * L, L)] << 1) + hl
                        for m in range(num_cb)
                    ]
                    himask = jnp.full((L,), -65536, jnp.int32)
                    for jp in range(igs // 2):
                        acc_lo = jnp.zeros((L,), jnp.float32)
                        acc_hi = jnp.zeros((L,), jnp.float32)
                        for m in range(num_cb):
                            off = (jp * num_cb + m) * cb_size * 2
                            w = plsc.load_gather(
                                cb_v.at[pl.ds(off, cb_size * 2)], [cvecs[m]])
                            acc_lo = acc_lo + plsc.bitcast(w << 16, jnp.float32)
                            acc_hi = acc_hi + plsc.bitcast(w & himask, jnp.float32)
                        plsc.store_scatter(out_blk, [scat_pats[2 * jp]],
                                           acc_lo * s)
                        plsc.store_scatter(out_blk, [scat_pats[2 * jp + 1]],
                                           acc_hi * s)

                cp_out(row, b).start()

                @pl.when(p < pairs - 1)
                def _next_in():
                    cp_in(row + 2, b).start()

            return carry

        lax.fori_loop(0, pairs, pair_body, 0)
        cp_out(row0 + rows_per_w - 2, 0).wait()
        cp_out(row0 + rows_per_w - 1, 1).wait()

    return k


def kernel(codes, codebooks, scales):
    num_out, num_in, num_cb = codes.shape
    _, cb_size, ogs, igs = codebooks.shape
    codes2d = codes.transpose(0, 2, 1).reshape(num_out, num_cb * num_in)
    cbj = codebooks.reshape(num_cb, cb_size, igs)
    bits = jax.lax.bitcast_convert_type(
        cbj.astype(jnp.bfloat16), jnp.uint16).astype(jnp.uint32)
    words = bits[:, :, 0::2] | (bits[:, :, 1::2] << 16)
    words2 = jnp.repeat(words.transpose(2, 0, 1).reshape(-1), 2)
    flat_cb = jax.lax.bitcast_convert_type(words2, jnp.int32)
    scales1d = scales.reshape(num_out)
    k = _make_sc_kernel(num_out, num_in, num_cb, igs, cb_size)
    return k(codes2d, flat_cb, scales1d)


# speedup vs baseline: 1.4813x; 1.0795x over previous
import functools

import jax
import jax.numpy as jnp
from jax import lax
from jax.experimental import pallas as pl
from jax.experimental.pallas import tpu as pltpu
from jax.experimental.pallas import tpu_sc as plsc


def _make_sc_kernel(num_out, num_in, num_cb, igs, cb_size):
    info = plsc.get_sparse_core_info()
    nc, ns, L = info.num_cores, info.num_subcores, info.num_lanes
    nw = nc * ns
    rows_per_w = num_out // nw
    flat_cb_len = num_cb * cb_size * igs
    num_in_elems = num_in * igs
    codes_row_len = num_in * num_cb
    blocks = num_in // L

    mesh = plsc.VectorSubcoreMesh(core_axis_name="c", subcore_axis_name="s")

    @functools.partial(
        pl.kernel,
        mesh=mesh,
        out_type=jax.ShapeDtypeStruct((num_out, num_in_elems), jnp.float32),
        scratch_types=[
            pltpu.VMEM((flat_cb_len // 2,), jnp.int32),
            pltpu.VMEM((rows_per_w,), jnp.float32),
            pltpu.VMEM((2, codes_row_len), jnp.int32),
            pltpu.VMEM((2, num_in_elems), jnp.float32),
            pltpu.SemaphoreType.DMA,
            pltpu.SemaphoreType.DMA,
            pltpu.SemaphoreType.DMA,
            pltpu.SemaphoreType.DMA,
        ],
        compiler_params=pltpu.CompilerParams(needs_layout_passes=False),
    )
    def k(codes_hbm, cb_hbm, scales_hbm, out_hbm, cb_v, sc_v, codes_v, out_v,
          si0, si1, so0, so1):
        wid = lax.axis_index("s") * nc + lax.axis_index("c")
        row0 = wid * rows_per_w
        pltpu.sync_copy(cb_hbm, cb_v)
        pltpu.sync_copy(scales_hbm.at[pl.ds(row0, rows_per_w)], sc_v)

        lane = lax.iota(jnp.int32, L)
        scat_pats = [lane * igs + j for j in range(igs)]
        sis = (si0, si1)
        sos = (so0, so1)

        def cp_in(row, b):
            return pltpu.make_async_copy(
                codes_hbm.at[row], codes_v.at[b], sis[b])

        def cp_out(row, b):
            return pltpu.make_async_copy(
                out_v.at[b], out_hbm.at[row], sos[b])

        cp_in(row0, 0).start()
        cp_in(row0 + 1, 1).start()
        pairs = rows_per_w // 2

        def pair_body(p, carry):
            for b in range(2):
                r = p * 2 + b
                row = row0 + r
                cp_in(row, b).wait()

                @pl.when(p > 0)
                def _wait_out():
                    cp_out(row - 2, b).wait()

                s = plsc.load_gather(sc_v, [jnp.full((L,), r, jnp.int32)])

                @plsc.parallel_loop(0, blocks, unroll=2)
                def blk_body(ib):
                    out_blk = out_v.at[b].at[pl.ds(ib * (L * igs), L * igs)]
                    cvecs = [
                        codes_v[b, pl.ds(m * num_in + ib * L, L)]
                        for m in range(num_cb)
                    ]
                    for jp in range(igs // 2):
                        acc_lo = jnp.zeros((L,), jnp.float32)
                        acc_hi = jnp.zeros((L,), jnp.float32)
                        for m in range(num_cb):
                            off = (jp * num_cb + m) * cb_size
                            w = plsc.load_gather(
                                cb_v.at[pl.ds(off, cb_size)], [cvecs[m]])
                            acc_lo = acc_lo + plsc.bitcast(w << 16, jnp.float32)
                            acc_hi = acc_hi + plsc.bitcast(w, jnp.float32)
                        plsc.store_scatter(out_blk, [scat_pats[2 * jp]],
                                           acc_lo * s)
                        plsc.store_scatter(out_blk, [scat_pats[2 * jp + 1]],
                                           acc_hi * s)

                cp_out(row, b).start()

                @pl.when(p < pairs - 1)
                def _next_in():
                    cp_in(row + 2, b).start()

            return carry

        lax.fori_loop(0, pairs, pair_body, 0)
        cp_out(row0 + rows_per_w - 2, 0).wait()
        cp_out(row0 + rows_per_w - 1, 1).wait()

    return k


def kernel(codes, codebooks, scales):
    num_out, num_in, num_cb = codes.shape
    _, cb_size, ogs, igs = codebooks.shape
    codes2d = codes.transpose(0, 2, 1).reshape(num_out, num_cb * num_in)
    cbj = codebooks.reshape(num_cb, cb_size, igs)
    bits = jax.lax.bitcast_convert_type(
        cbj.astype(jnp.bfloat16), jnp.uint16).astype(jnp.uint32)
    words = bits[:, :, 0::2] | (bits[:, :, 1::2] << 16)
    flat_cb = jax.lax.bitcast_convert_type(
        words.transpose(2, 0, 1).reshape(-1), jnp.int32)
    scales1d = scales.reshape(num_out)
    k = _make_sc_kernel(num_out, num_in, num_cb, igs, cb_size)
    return k(codes2d, flat_cb, scales1d)
